# baseline (speedup 1.0000x reference)
import functools

import jax
import jax.numpy as jnp
from jax.experimental import pallas as pl
from jax.experimental.pallas import tpu as pltpu


def _pick_tile(n, candidates):
    for c in candidates:
        if n % c == 0:
            return c
    return n


def _yw_body(x_ref, w1_ref, w2_ref, y1_ref, y2_ref):
    x = x_ref[...].astype(jnp.bfloat16)
    w1 = w1_ref[...].astype(jnp.bfloat16)
    w2 = w2_ref[...].astype(jnp.bfloat16)
    dn = (((1,), (0,)), ((), ()))
    y1_ref[...] = jax.lax.dot_general(
        x, w1, dn, preferred_element_type=jnp.float32).astype(jnp.bfloat16)
    y2_ref[...] = jax.lax.dot_general(
        x, w2, dn, preferred_element_type=jnp.float32).astype(jnp.bfloat16)


def _main_body(ti, tj, a_ref, cm_ref, rm_ref, y1_ref, y2t_ref,
               out_in_ref, out_outt_ref, deg_r_ref, deg_c_ref):
    j = pl.program_id(0)
    i = pl.program_id(1)
    a = a_ref[...]
    cm = cm_ref[...].reshape(1, tj)
    rm = rm_ref[...]
    a = jnp.where(cm > 0.5, a, 0.0)
    a = jnp.where(rm > 0.5, a, 0.0)
    ab = a.astype(jnp.bfloat16)
    y1 = y1_ref[...]
    isl = pl.ds(i * ti, ti)
    y2t = y2t_ref[:, isl]

    c_in = jax.lax.dot_general(
        ab, y1, (((1,), (0,)), ((), ())), preferred_element_type=jnp.float32)
    c_outt = jax.lax.dot_general(
        y2t, ab, (((1,), (0,)), ((), ())), preferred_element_type=jnp.float32)
    rs = jnp.sum(a, axis=1, keepdims=True)
    cs = jnp.sum(a, axis=0, keepdims=True).reshape(1, 1, tj)

    @pl.when(j == 0)
    def _():
        out_in_ref[isl, :] = c_in
        deg_r_ref[isl, :] = rs

    @pl.when(j > 0)
    def _():
        out_in_ref[isl, :] += c_in
        deg_r_ref[isl, :] += rs

    @pl.when(i == 0)
    def _():
        out_outt_ref[...] = c_outt
        deg_c_ref[...] = cs

    @pl.when(i > 0)
    def _():
        out_outt_ref[...] += c_outt
        deg_c_ref[...] += cs


def _epi_body(oi_ref, oo_ref, dr_ref, dc_ref, out_ref):
    r1 = 1.0 / jnp.clip(dr_ref[...], 1e-6, None)
    r2 = 1.0 / jnp.clip(dc_ref[...], 1e-6, None)
    out_ref[...] = oi_ref[...] * r1 + oo_ref[...] * r2


def kernel(X, A, W_in, W_out):
    n, d_in = X.shape
    d_out = W_in.shape[1]

    tb = _pick_tile(n, (2000, 1000, 400, 200, 80, 40, 16, 8))
    y1, y2 = pl.pallas_call(
        _yw_body,
        grid=(n // tb,),
        in_specs=[
            pl.BlockSpec((tb, d_in), lambda b: (b, 0)),
            pl.BlockSpec((d_in, d_out), lambda b: (0, 0)),
            pl.BlockSpec((d_in, d_out), lambda b: (0, 0)),
        ],
        out_specs=[
            pl.BlockSpec((tb, d_out), lambda b: (b, 0)),
            pl.BlockSpec((tb, d_out), lambda b: (b, 0)),
        ],
        out_shape=[
            jax.ShapeDtypeStruct((n, d_out), jnp.bfloat16),
            jax.ShapeDtypeStruct((n, d_out), jnp.bfloat16),
        ],
    )(X, W_in, W_out)

    ti = tj = 1024 if n >= 1024 else n
    ni = nj = -(-n // ti)
    n_pad = ni * ti
    if n_pad != n:
        pad = ((0, n_pad - n), (0, 0))
        y1 = jnp.pad(y1, pad)
    y2t = jnp.pad(y2.T, ((0, 0), (0, n_pad - n)))
    valid = jnp.pad(jnp.ones((n,), jnp.float32), (0, n_pad - n))
    col_valid = valid.reshape(nj, 1, tj)
    row_valid = valid.reshape(n_pad, 1)
    out_in, out_outt, deg_r, deg_c = pl.pallas_call(
        functools.partial(_main_body, ti, tj),
        grid=(nj, ni),
        in_specs=[
            pl.BlockSpec((ti, tj), lambda j, i: (i, j)),
            pl.BlockSpec((1, 1, tj), lambda j, i: (j, 0, 0)),
            pl.BlockSpec((ti, 1), lambda j, i: (i, 0)),
            pl.BlockSpec((tj, d_out), lambda j, i: (j, 0)),
            pl.BlockSpec((d_out, n_pad), lambda j, i: (0, 0)),
        ],
        out_specs=[
            pl.BlockSpec((n_pad, d_out), lambda j, i: (0, 0)),
            pl.BlockSpec((d_out, tj), lambda j, i: (0, j)),
            pl.BlockSpec((n_pad, 1), lambda j, i: (0, 0)),
            pl.BlockSpec((1, 1, tj), lambda j, i: (j, 0, 0)),
        ],
        out_shape=[
            jax.ShapeDtypeStruct((n_pad, d_out), jnp.float32),
            jax.ShapeDtypeStruct((d_out, n_pad), jnp.float32),
            jax.ShapeDtypeStruct((n_pad, 1), jnp.float32),
            jax.ShapeDtypeStruct((nj, 1, tj), jnp.float32),
        ],
        compiler_params=pltpu.CompilerParams(
            dimension_semantics=("arbitrary", "arbitrary"),
            vmem_limit_bytes=64 * 1024 * 1024,
        ),
    )(A, col_valid, row_valid, y1, y2t)
    out_out = out_outt.T

    deg_c_col = deg_c.reshape(n_pad, 1)

    te = ti
    out = pl.pallas_call(
        _epi_body,
        grid=(n_pad // te,),
        in_specs=[
            pl.BlockSpec((te, d_out), lambda b: (b, 0)),
            pl.BlockSpec((te, d_out), lambda b: (b, 0)),
            pl.BlockSpec((te, 1), lambda b: (b, 0)),
            pl.BlockSpec((te, 1), lambda b: (b, 0)),
        ],
        out_specs=pl.BlockSpec((te, d_out), lambda b: (b, 0)),
        out_shape=jax.ShapeDtypeStruct((n_pad, d_out), jnp.float32),
    )(out_in, out_out, deg_r, deg_c_col)
    return out[:n]


# speedup vs baseline: 1.0905x; 1.0905x over previous
import functools

import jax
import jax.numpy as jnp
from jax.experimental import pallas as pl
from jax.experimental.pallas import tpu as pltpu


def _pick_tile(n, candidates):
    for c in candidates:
        if n % c == 0:
            return c
    return n


def _yw_body(x_ref, w1_ref, w2_ref, y1_ref, y2_ref):
    x = x_ref[...].astype(jnp.bfloat16)
    w1 = w1_ref[...].astype(jnp.bfloat16)
    w2 = w2_ref[...].astype(jnp.bfloat16)
    dn = (((1,), (0,)), ((), ()))
    y1_ref[...] = jax.lax.dot_general(
        x, w1, dn, preferred_element_type=jnp.float32).astype(jnp.bfloat16)
    y2_ref[...] = jax.lax.dot_general(
        x, w2, dn, preferred_element_type=jnp.float32).astype(jnp.bfloat16)


def _main_body(ti, tj, a_ref, cm_ref, rm_ref, y1_ref, y2t_ref,
               out_in_ref, out_outt_ref, deg_r_ref, deg_c_ref):
    j = pl.program_id(0)
    i = pl.program_id(1)
    a = a_ref[...]
    cm = cm_ref[...].reshape(1, tj)
    rm = rm_ref[...]
    ab = a.astype(jnp.bfloat16)
    y1 = y1_ref[...]
    isl = pl.ds(i * ti, ti)
    y2t = y2t_ref[:, isl]

    c_in = jax.lax.dot_general(
        ab, y1, (((1,), (0,)), ((), ())), preferred_element_type=jnp.float32)
    c_outt = jax.lax.dot_general(
        y2t, ab, (((1,), (0,)), ((), ())), preferred_element_type=jnp.float32)
    rs = jnp.sum(a * cm, axis=1, keepdims=True)
    cs = jnp.sum(a * rm, axis=0, keepdims=True).reshape(1, 1, tj)

    @pl.when(j == 0)
    def _():
        out_in_ref[isl, :] = c_in
        deg_r_ref[isl, :] = rs

    @pl.when(j > 0)
    def _():
        out_in_ref[isl, :] += c_in
        deg_r_ref[isl, :] += rs

    @pl.when(i == 0)
    def _():
        out_outt_ref[...] = c_outt
        deg_c_ref[...] = cs

    @pl.when(i > 0)
    def _():
        out_outt_ref[...] += c_outt
        deg_c_ref[...] += cs


def _epi_body(oi_ref, oo_ref, dr_ref, dc_ref, out_ref):
    r1 = 1.0 / jnp.clip(dr_ref[...], 1e-6, None)
    r2 = 1.0 / jnp.clip(dc_ref[...], 1e-6, None)
    out_ref[...] = oi_ref[...] * r1 + oo_ref[...] * r2


def kernel(X, A, W_in, W_out):
    n, d_in = X.shape
    d_out = W_in.shape[1]

    tb = _pick_tile(n, (2000, 1000, 400, 200, 80, 40, 16, 8))
    y1, y2 = pl.pallas_call(
        _yw_body,
        grid=(n // tb,),
        in_specs=[
            pl.BlockSpec((tb, d_in), lambda b: (b, 0)),
            pl.BlockSpec((d_in, d_out), lambda b: (0, 0)),
            pl.BlockSpec((d_in, d_out), lambda b: (0, 0)),
        ],
        out_specs=[
            pl.BlockSpec((tb, d_out), lambda b: (b, 0)),
            pl.BlockSpec((tb, d_out), lambda b: (b, 0)),
        ],
        out_shape=[
            jax.ShapeDtypeStruct((n, d_out), jnp.bfloat16),
            jax.ShapeDtypeStruct((n, d_out), jnp.bfloat16),
        ],
    )(X, W_in, W_out)

    ti = tj = 1024 if n >= 1024 else n
    ni = nj = -(-n // ti)
    n_pad = ni * ti
    if n_pad != n:
        pad = ((0, n_pad - n), (0, 0))
        y1 = jnp.pad(y1, pad)
    y2t = jnp.pad(y2.T, ((0, 0), (0, n_pad - n)))
    valid = jnp.pad(jnp.ones((n,), jnp.float32), (0, n_pad - n))
    col_valid = valid.reshape(nj, 1, tj)
    row_valid = valid.reshape(n_pad, 1)
    out_in, out_outt, deg_r, deg_c = pl.pallas_call(
        functools.partial(_main_body, ti, tj),
        grid=(nj, ni),
        in_specs=[
            pl.BlockSpec((ti, tj), lambda j, i: (i, j)),
            pl.BlockSpec((1, 1, tj), lambda j, i: (j, 0, 0)),
            pl.BlockSpec((ti, 1), lambda j, i: (i, 0)),
            pl.BlockSpec((tj, d_out), lambda j, i: (j, 0)),
            pl.BlockSpec((d_out, n_pad), lambda j, i: (0, 0)),
        ],
        out_specs=[
            pl.BlockSpec((n_pad, d_out), lambda j, i: (0, 0)),
            pl.BlockSpec((d_out, tj), lambda j, i: (0, j)),
            pl.BlockSpec((n_pad, 1), lambda j, i: (0, 0)),
            pl.BlockSpec((1, 1, tj), lambda j, i: (j, 0, 0)),
        ],
        out_shape=[
            jax.ShapeDtypeStruct((n_pad, d_out), jnp.float32),
            jax.ShapeDtypeStruct((d_out, n_pad), jnp.float32),
            jax.ShapeDtypeStruct((n_pad, 1), jnp.float32),
            jax.ShapeDtypeStruct((nj, 1, tj), jnp.float32),
        ],
        compiler_params=pltpu.CompilerParams(
            dimension_semantics=("arbitrary", "arbitrary"),
            vmem_limit_bytes=64 * 1024 * 1024,
        ),
    )(A, col_valid, row_valid, y1, y2t)
    out_out = out_outt.T

    deg_c_col = deg_c.reshape(n_pad, 1)

    te = ti
    out = pl.pallas_call(
        _epi_body,
        grid=(n_pad // te,),
        in_specs=[
            pl.BlockSpec((te, d_out), lambda b: (b, 0)),
            pl.BlockSpec((te, d_out), lambda b: (b, 0)),
            pl.BlockSpec((te, 1), lambda b: (b, 0)),
            pl.BlockSpec((te, 1), lambda b: (b, 0)),
        ],
        out_specs=pl.BlockSpec((te, d_out), lambda b: (b, 0)),
        out_shape=jax.ShapeDtypeStruct((n_pad, d_out), jnp.float32),
    )(out_in, out_out, deg_r, deg_c_col)
    return out[:n]


# speedup vs baseline: 1.2311x; 1.1289x over previous
import functools

import jax
import jax.numpy as jnp
from jax.experimental import pallas as pl
from jax.experimental.pallas import tpu as pltpu


def _pick_tile(n, candidates):
    for c in candidates:
        if n % c == 0:
            return c
    return n


def _yw_body(x_ref, rm_ref, w1_ref, w2_ref, y1_ref, y2t_ref):
    rm = rm_ref[...]
    x = jnp.where(rm > 0.5, x_ref[...], 0.0).astype(jnp.bfloat16)
    w1 = w1_ref[...].astype(jnp.bfloat16)
    w2 = w2_ref[...].astype(jnp.bfloat16)
    y1_ref[...] = jax.lax.dot_general(
        x, w1, (((1,), (0,)), ((), ())),
        preferred_element_type=jnp.float32).astype(jnp.bfloat16)
    y2t_ref[...] = jax.lax.dot_general(
        w2, x, (((0,), (1,)), ((), ())),
        preferred_element_type=jnp.float32).astype(jnp.bfloat16)


def _main_body(ti, tj, a_ref, cm_ref, rm_ref, y1_ref, y2t_ref,
               out_in_ref, out_outt_ref, deg_r_ref, deg_c_ref):
    j = pl.program_id(0)
    i = pl.program_id(1)
    a = a_ref[...]
    cm = cm_ref[...].reshape(1, tj)
    rm = rm_ref[...]
    ab = a.astype(jnp.bfloat16)
    y1 = y1_ref[...]
    isl = pl.ds(i * ti, ti)
    y2t = y2t_ref[:, isl]

    c_in = jax.lax.dot_general(
        ab, y1, (((1,), (0,)), ((), ())), preferred_element_type=jnp.float32)
    c_outt = jax.lax.dot_general(
        y2t, ab, (((1,), (0,)), ((), ())), preferred_element_type=jnp.float32)
    rs = jnp.sum(a * cm, axis=1, keepdims=True)
    cs = jnp.sum(a * rm, axis=0, keepdims=True).reshape(1, 1, tj)

    @pl.when(j == 0)
    def _():
        out_in_ref[isl, :] = c_in
        deg_r_ref[isl, :] = rs

    @pl.when(j > 0)
    def _():
        out_in_ref[isl, :] += c_in
        deg_r_ref[isl, :] += rs

    @pl.when(i == 0)
    def _():
        out_outt_ref[...] = c_outt
        deg_c_ref[...] = cs

    @pl.when(i > 0)
    def _():
        out_outt_ref[...] += c_outt
        deg_c_ref[...] += cs


def _epi_body(te, oi_ref, oot_ref, dr_ref, dc_ref, out_ref):
    r1 = 1.0 / jnp.clip(dr_ref[...], 1e-6, None)
    r2 = 1.0 / jnp.clip(dc_ref[...].reshape(1, te), 1e-6, None)
    oot = oot_ref[...] * r2
    out_ref[...] = oi_ref[...] * r1 + oot.T


def kernel(X, A, W_in, W_out):
    n, d_in = X.shape
    d_out = W_in.shape[1]

    ti = tj = 1024 if n >= 1024 else n
    ni = nj = -(-n // ti)
    n_pad = ni * ti
    valid = jnp.pad(jnp.ones((n,), jnp.float32), (0, n_pad - n))
    col_valid = valid.reshape(nj, 1, tj)
    row_valid = valid.reshape(n_pad, 1)

    y1, y2t = pl.pallas_call(
        _yw_body,
        grid=(ni,),
        in_specs=[
            pl.BlockSpec((ti, d_in), lambda b: (b, 0)),
            pl.BlockSpec((ti, 1), lambda b: (b, 0)),
            pl.BlockSpec((d_in, d_out), lambda b: (0, 0)),
            pl.BlockSpec((d_in, d_out), lambda b: (0, 0)),
        ],
        out_specs=[
            pl.BlockSpec((ti, d_out), lambda b: (b, 0)),
            pl.BlockSpec((d_out, ti), lambda b: (0, b)),
        ],
        out_shape=[
            jax.ShapeDtypeStruct((n_pad, d_out), jnp.bfloat16),
            jax.ShapeDtypeStruct((d_out, n_pad), jnp.bfloat16),
        ],
    )(X, row_valid, W_in, W_out)

    out_in, out_outt, deg_r, deg_c = pl.pallas_call(
        functools.partial(_main_body, ti, tj),
        grid=(nj, ni),
        in_specs=[
            pl.BlockSpec((ti, tj), lambda j, i: (i, j)),
            pl.BlockSpec((1, 1, tj), lambda j, i: (j, 0, 0)),
            pl.BlockSpec((ti, 1), lambda j, i: (i, 0)),
            pl.BlockSpec((tj, d_out), lambda j, i: (j, 0)),
            pl.BlockSpec((d_out, n_pad), lambda j, i: (0, 0)),
        ],
        out_specs=[
            pl.BlockSpec((n_pad, d_out), lambda j, i: (0, 0)),
            pl.BlockSpec((d_out, tj), lambda j, i: (0, j)),
            pl.BlockSpec((n_pad, 1), lambda j, i: (0, 0)),
            pl.BlockSpec((1, 1, tj), lambda j, i: (j, 0, 0)),
        ],
        out_shape=[
            jax.ShapeDtypeStruct((n_pad, d_out), jnp.float32),
            jax.ShapeDtypeStruct((d_out, n_pad), jnp.float32),
            jax.ShapeDtypeStruct((n_pad, 1), jnp.float32),
            jax.ShapeDtypeStruct((nj, 1, tj), jnp.float32),
        ],
        compiler_params=pltpu.CompilerParams(
            dimension_semantics=("arbitrary", "arbitrary"),
            vmem_limit_bytes=64 * 1024 * 1024,
        ),
    )(A, col_valid, row_valid, y1, y2t)

    te = ti
    out = pl.pallas_call(
        functools.partial(_epi_body, te),
        grid=(n_pad // te,),
        in_specs=[
            pl.BlockSpec((te, d_out), lambda b: (b, 0)),
            pl.BlockSpec((d_out, te), lambda b: (0, b)),
            pl.BlockSpec((te, 1), lambda b: (b, 0)),
            pl.BlockSpec((1, 1, te), lambda b: (b, 0, 0)),
        ],
        out_specs=pl.BlockSpec((te, d_out), lambda b: (b, 0)),
        out_shape=jax.ShapeDtypeStruct((n_pad, d_out), jnp.float32),
    )(out_in, out_outt, deg_r, deg_c)
    return out[:n]


# speedup vs baseline: 1.2632x; 1.0261x over previous
import functools

import jax
import jax.numpy as jnp
from jax.experimental import pallas as pl
from jax.experimental.pallas import tpu as pltpu


def _pick_tile(n, candidates):
    for c in candidates:
        if n % c == 0:
            return c
    return n


def _yw_body(x_ref, rm_ref, w1_ref, w2_ref, y1_ref, y2t_ref):
    rm = rm_ref[...]
    x = jnp.where(rm > 0.5, x_ref[...], 0.0).astype(jnp.bfloat16)
    w1 = w1_ref[...].astype(jnp.bfloat16)
    w2 = w2_ref[...].astype(jnp.bfloat16)
    y1_ref[...] = jax.lax.dot_general(
        x, w1, (((1,), (0,)), ((), ())),
        preferred_element_type=jnp.float32).astype(jnp.bfloat16)
    y2t_ref[...] = jax.lax.dot_general(
        w2, x, (((0,), (1,)), ((), ())),
        preferred_element_type=jnp.float32).astype(jnp.bfloat16)


def _main_body(ti, tj, a_ref, cm_ref, y1_ref, y2t_ref,
               out_in_ref, out_outt_ref, deg_r_ref):
    j = pl.program_id(0)
    i = pl.program_id(1)
    a = a_ref[...]
    cm = cm_ref[...].reshape(1, tj)
    ab = a.astype(jnp.bfloat16)
    y1 = y1_ref[...]
    isl = pl.ds(i * ti, ti)
    y2t = y2t_ref[:, isl]

    c_in = jax.lax.dot_general(
        ab, y1, (((1,), (0,)), ((), ())), preferred_element_type=jnp.float32)
    c_outt = jax.lax.dot_general(
        y2t, ab, (((1,), (0,)), ((), ())), preferred_element_type=jnp.float32)
    rs = jnp.sum(a * cm, axis=1, keepdims=True)

    @pl.when(j == 0)
    def _():
        out_in_ref[isl, :] = c_in
        deg_r_ref[isl, :] = rs

    @pl.when(j > 0)
    def _():
        out_in_ref[isl, :] += c_in
        deg_r_ref[isl, :] += rs

    @pl.when(i == 0)
    def _():
        out_outt_ref[...] = c_outt

    @pl.when(i > 0)
    def _():
        out_outt_ref[...] += c_outt


def _epi_body(d_out, oi_ref, oot_ref, dr_ref, out_ref):
    r1 = 1.0 / jnp.clip(dr_ref[...], 1e-6, None)
    oota = oot_ref[...]
    r2 = 1.0 / jnp.clip(oota[d_out:, :], 1e-6, None)
    oot = oota[:d_out, :] * r2
    out_ref[...] = oi_ref[...] * r1 + oot.T


def kernel(X, A, W_in, W_out):
    n, d_in = X.shape
    d_out = W_in.shape[1]

    ti = tj = 1024 if n >= 1024 else n
    ni = nj = -(-n // ti)
    n_pad = ni * ti
    valid = jnp.pad(jnp.ones((n,), jnp.float32), (0, n_pad - n))
    col_valid = valid.reshape(nj, 1, tj)
    row_valid = valid.reshape(n_pad, 1)

    y1, y2t = pl.pallas_call(
        _yw_body,
        grid=(ni,),
        in_specs=[
            pl.BlockSpec((ti, d_in), lambda b: (b, 0)),
            pl.BlockSpec((ti, 1), lambda b: (b, 0)),
            pl.BlockSpec((d_in, d_out), lambda b: (0, 0)),
            pl.BlockSpec((d_in, d_out), lambda b: (0, 0)),
        ],
        out_specs=[
            pl.BlockSpec((ti, d_out), lambda b: (b, 0)),
            pl.BlockSpec((d_out, ti), lambda b: (0, b)),
        ],
        out_shape=[
            jax.ShapeDtypeStruct((n_pad, d_out), jnp.bfloat16),
            jax.ShapeDtypeStruct((d_out, n_pad), jnp.bfloat16),
        ],
    )(X, row_valid, W_in, W_out)

    y2t_aug = jnp.concatenate(
        [y2t, valid.reshape(1, n_pad).astype(jnp.bfloat16)], axis=0)
    out_in, out_outt, deg_r = pl.pallas_call(
        functools.partial(_main_body, ti, tj),
        grid=(nj, ni),
        in_specs=[
            pl.BlockSpec((ti, tj), lambda j, i: (i, j)),
            pl.BlockSpec((1, 1, tj), lambda j, i: (j, 0, 0)),
            pl.BlockSpec((tj, d_out), lambda j, i: (j, 0)),
            pl.BlockSpec((d_out + 1, n_pad), lambda j, i: (0, 0)),
        ],
        out_specs=[
            pl.BlockSpec((n_pad, d_out), lambda j, i: (0, 0)),
            pl.BlockSpec((d_out + 1, tj), lambda j, i: (0, j)),
            pl.BlockSpec((n_pad, 1), lambda j, i: (0, 0)),
        ],
        out_shape=[
            jax.ShapeDtypeStruct((n_pad, d_out), jnp.float32),
            jax.ShapeDtypeStruct((d_out + 1, n_pad), jnp.float32),
            jax.ShapeDtypeStruct((n_pad, 1), jnp.float32),
        ],
        compiler_params=pltpu.CompilerParams(
            dimension_semantics=("arbitrary", "arbitrary"),
            vmem_limit_bytes=64 * 1024 * 1024,
        ),
    )(A, col_valid, y1, y2t_aug)

    te = ti
    out = pl.pallas_call(
        functools.partial(_epi_body, d_out),
        grid=(n_pad // te,),
        in_specs=[
            pl.BlockSpec((te, d_out), lambda b: (b, 0)),
            pl.BlockSpec((d_out + 1, te), lambda b: (0, b)),
            pl.BlockSpec((te, 1), lambda b: (b, 0)),
        ],
        out_specs=pl.BlockSpec((te, d_out), lambda b: (b, 0)),
        out_shape=jax.ShapeDtypeStruct((n_pad, d_out), jnp.float32),
    )(out_in, out_outt, deg_r)
    return out[:n]


# speedup vs baseline: 1.3471x; 1.0664x over previous
import functools

import jax
import jax.numpy as jnp
from jax.experimental import pallas as pl
from jax.experimental.pallas import tpu as pltpu


def _pick_tile(n, candidates):
    for c in candidates:
        if n % c == 0:
            return c
    return n


def _yw_body(x_ref, rm_ref, w1_ref, w2_ref, y1_ref, y2t_ref):
    rm = rm_ref[...]
    x = jnp.where(rm > 0.5, x_ref[...], 0.0).astype(jnp.bfloat16)
    w1 = w1_ref[...].astype(jnp.bfloat16)
    w2 = w2_ref[...].astype(jnp.bfloat16)
    y1_ref[...] = jax.lax.dot_general(
        x, w1, (((1,), (0,)), ((), ())),
        preferred_element_type=jnp.float32).astype(jnp.bfloat16)
    y2t_ref[...] = jax.lax.dot_general(
        w2, x, (((0,), (1,)), ((), ())),
        preferred_element_type=jnp.float32).astype(jnp.bfloat16)


def _main_body(ti, tj, a_ref, cm_ref, y1_ref, y2t_ref,
               out_in_ref, out_outt_ref, deg_r_ref):
    j = pl.program_id(0)
    i = pl.program_id(1)
    cm = cm_ref[...].reshape(1, tj)
    a = a_ref[...]
    ab = a.astype(jnp.bfloat16)
    y1 = y1_ref[...]
    isl = pl.ds(i * ti, ti)
    y2t = y2t_ref[...]

    c_in = jax.lax.dot_general(
        ab, y1, (((1,), (0,)), ((), ())), preferred_element_type=jnp.float32)
    c_outt = jax.lax.dot_general(
        y2t, ab, (((1,), (0,)), ((), ())), preferred_element_type=jnp.float32)
    rs = jnp.sum(a * cm, axis=1, keepdims=True)

    @pl.when(j == 0)
    def _():
        out_in_ref[isl, :] = c_in
        deg_r_ref[isl, :] = rs

    @pl.when(j > 0)
    def _():
        out_in_ref[isl, :] += c_in
        deg_r_ref[isl, :] += rs

    @pl.when(i == 0)
    def _():
        out_outt_ref[...] = c_outt

    @pl.when(i > 0)
    def _():
        out_outt_ref[...] += c_outt


def _epi_body(d_out, oi_ref, oot_ref, dr_ref, out_ref):
    r1 = 1.0 / jnp.clip(dr_ref[...], 1e-6, None)
    oota = oot_ref[...]
    r2 = 1.0 / jnp.clip(oota[d_out:, :], 1e-6, None)
    oot = oota[:d_out, :] * r2
    out_ref[...] = oi_ref[...] * r1 + oot.T


def kernel(X, A, W_in, W_out):
    n, d_in = X.shape
    d_out = W_in.shape[1]

    if n >= 2048:
        ti, tj = 2048, 1024
    else:
        ti = tj = n
    nj = -(-n // tj)
    ni = -(-n // ti)
    n_pad = nj * tj
    assert ni * ti == n_pad
    valid = jnp.pad(jnp.ones((n,), jnp.float32), (0, n_pad - n))
    col_valid = valid.reshape(nj, 1, tj)
    row_valid = valid.reshape(n_pad, 1)

    y1, y2t = pl.pallas_call(
        _yw_body,
        grid=(ni,),
        in_specs=[
            pl.BlockSpec((ti, d_in), lambda b: (b, 0)),
            pl.BlockSpec((ti, 1), lambda b: (b, 0)),
            pl.BlockSpec((d_in, d_out), lambda b: (0, 0)),
            pl.BlockSpec((d_in, d_out), lambda b: (0, 0)),
        ],
        out_specs=[
            pl.BlockSpec((ti, d_out), lambda b: (b, 0)),
            pl.BlockSpec((d_out, ti), lambda b: (0, b)),
        ],
        out_shape=[
            jax.ShapeDtypeStruct((n_pad, d_out), jnp.bfloat16),
            jax.ShapeDtypeStruct((d_out, n_pad), jnp.bfloat16),
        ],
    )(X, row_valid, W_in, W_out)

    y2t_aug = jnp.concatenate(
        [y2t, valid.reshape(1, n_pad).astype(jnp.bfloat16)], axis=0)
    out_in, out_outt, deg_r = pl.pallas_call(
        functools.partial(_main_body, ti, tj),
        grid=(nj, ni),
        in_specs=[
            pl.BlockSpec((ti, tj), lambda j, i: (i, j)),
            pl.BlockSpec((1, 1, tj), lambda j, i: (j, 0, 0)),
            pl.BlockSpec((tj, d_out), lambda j, i: (j, 0)),
            pl.BlockSpec((d_out + 1, ti), lambda j, i: (0, i)),
        ],
        out_specs=[
            pl.BlockSpec((n_pad, d_out), lambda j, i: (0, 0)),
            pl.BlockSpec((d_out + 1, tj), lambda j, i: (0, j)),
            pl.BlockSpec((n_pad, 1), lambda j, i: (0, 0)),
        ],
        out_shape=[
            jax.ShapeDtypeStruct((n_pad, d_out), jnp.float32),
            jax.ShapeDtypeStruct((d_out + 1, n_pad), jnp.float32),
            jax.ShapeDtypeStruct((n_pad, 1), jnp.float32),
        ],
        compiler_params=pltpu.CompilerParams(
            dimension_semantics=("arbitrary", "arbitrary"),
            vmem_limit_bytes=64 * 1024 * 1024,
        ),
    )(A, col_valid, y1, y2t_aug)

    te = ti
    out = pl.pallas_call(
        functools.partial(_epi_body, d_out),
        grid=(n_pad // te,),
        in_specs=[
            pl.BlockSpec((te, d_out), lambda b: (b, 0)),
            pl.BlockSpec((d_out + 1, te), lambda b: (0, b)),
            pl.BlockSpec((te, 1), lambda b: (b, 0)),
        ],
        out_specs=pl.BlockSpec((te, d_out), lambda b: (b, 0)),
        out_shape=jax.ShapeDtypeStruct((n_pad, d_out), jnp.float32),
    )(out_in, out_outt, deg_r)
    return out[:n]


# speedup vs baseline: 1.4112x; 1.0475x over previous
import functools

import jax
import jax.numpy as jnp
from jax.experimental import pallas as pl
from jax.experimental.pallas import tpu as pltpu


def _pick_tile(n, candidates):
    for c in candidates:
        if n % c == 0:
            return c
    return n


def _yw_body(x_ref, rm_ref, w1_ref, w2_ref, y1_ref, y2t_ref):
    rm = rm_ref[...]
    x = jnp.where(rm > 0.5, x_ref[...], 0.0).astype(jnp.bfloat16)
    w1 = w1_ref[...].astype(jnp.bfloat16)
    w2 = w2_ref[...].astype(jnp.bfloat16)
    y1_ref[...] = jax.lax.dot_general(
        x, w1, (((1,), (0,)), ((), ())),
        preferred_element_type=jnp.float32).astype(jnp.bfloat16)
    y2t_ref[...] = jax.lax.dot_general(
        w2, x, (((0,), (1,)), ((), ())),
        preferred_element_type=jnp.float32).astype(jnp.bfloat16)


def _main_body(ti, tj, a_ref, cm_ref, y1_ref, y2t_ref,
               out_in_ref, out_outt_ref, deg_r_ref):
    j = pl.program_id(0)
    i = pl.program_id(1)
    cm = cm_ref[...].reshape(1, tj)
    a = a_ref[...]
    ab = a.astype(jnp.bfloat16)
    y1 = y1_ref[...]
    isl = pl.ds(i * ti, ti)
    y2t = y2t_ref[...]

    c_in = jax.lax.dot_general(
        ab, y1, (((1,), (0,)), ((), ())), preferred_element_type=jnp.float32)
    c_outt = jax.lax.dot_general(
        y2t, ab, (((1,), (0,)), ((), ())), preferred_element_type=jnp.float32)
    rs = jnp.sum(a * cm, axis=1, keepdims=True)

    @pl.when(j == 0)
    def _():
        out_in_ref[isl, :] = c_in
        deg_r_ref[isl, :] = rs

    @pl.when(j > 0)
    def _():
        out_in_ref[isl, :] += c_in
        deg_r_ref[isl, :] += rs

    @pl.when(i == 0)
    def _():
        out_outt_ref[...] = c_outt

    @pl.when(i > 0)
    def _():
        out_outt_ref[...] += c_outt


def _epi_body(d_out, oi_ref, oot_ref, dr_ref, out_ref):
    r1 = 1.0 / jnp.clip(dr_ref[...], 1e-6, None)
    oota = oot_ref[...]
    r2 = 1.0 / jnp.clip(oota[d_out:, :], 1e-6, None)
    oot = oota[:d_out, :] * r2
    out_ref[...] = oi_ref[...] * r1 + oot.T


def kernel(X, A, W_in, W_out):
    n, d_in = X.shape
    d_out = W_in.shape[1]

    if n >= 2048:
        ti, tj = 2048, 1024
    else:
        ti = tj = n
    nj = -(-n // tj)
    ni = -(-n // ti)
    n_pad = nj * tj
    assert ni * ti == n_pad
    valid = jnp.pad(jnp.ones((n,), jnp.float32), (0, n_pad - n))
    col_valid = valid.reshape(nj, 1, tj)
    row_valid = valid.reshape(n_pad, 1)

    y1, y2t = pl.pallas_call(
        _yw_body,
        grid=(ni,),
        in_specs=[
            pl.BlockSpec((ti, d_in), lambda b: (b, 0)),
            pl.BlockSpec((ti, 1), lambda b: (b, 0)),
            pl.BlockSpec((d_in, d_out), lambda b: (0, 0)),
            pl.BlockSpec((d_in, d_out), lambda b: (0, 0)),
        ],
        out_specs=[
            pl.BlockSpec((ti, d_out), lambda b: (b, 0)),
            pl.BlockSpec((d_out, ti), lambda b: (0, b)),
        ],
        out_shape=[
            jax.ShapeDtypeStruct((n_pad, d_out), jnp.bfloat16),
            jax.ShapeDtypeStruct((d_out, n_pad), jnp.bfloat16),
        ],
    )(X, row_valid, W_in, W_out)

    y2t_aug = jnp.concatenate(
        [y2t, valid.reshape(1, n_pad).astype(jnp.bfloat16)], axis=0)
    out_in, out_outt, deg_r = pl.pallas_call(
        functools.partial(_main_body, ti, tj),
        grid=(nj, ni),
        in_specs=[
            pl.BlockSpec((ti, tj), lambda j, i: (i, j)),
            pl.BlockSpec((1, 1, tj), lambda j, i: (j, 0, 0)),
            pl.BlockSpec((tj, d_out), lambda j, i: (j, 0)),
            pl.BlockSpec((d_out + 1, ti), lambda j, i: (0, i)),
        ],
        out_specs=[
            pl.BlockSpec((n_pad, d_out), lambda j, i: (0, 0)),
            pl.BlockSpec((d_out + 1, tj), lambda j, i: (0, j)),
            pl.BlockSpec((n_pad, 1), lambda j, i: (0, 0)),
        ],
        out_shape=[
            jax.ShapeDtypeStruct((n_pad, d_out), jnp.float32),
            jax.ShapeDtypeStruct((d_out + 1, n_pad), jnp.float32),
            jax.ShapeDtypeStruct((n_pad, 1), jnp.float32),
        ],
        compiler_params=pltpu.CompilerParams(
            dimension_semantics=("arbitrary", "arbitrary"),
            vmem_limit_bytes=64 * 1024 * 1024,
        ),
    )(A, col_valid, y1, y2t_aug)

    te = 1920 if n >= 2048 else n
    out = pl.pallas_call(
        functools.partial(_epi_body, d_out),
        grid=(-(-n // te),),
        in_specs=[
            pl.BlockSpec((te, d_out), lambda b: (b, 0)),
            pl.BlockSpec((d_out + 1, te), lambda b: (0, b)),
            pl.BlockSpec((te, 1), lambda b: (b, 0)),
        ],
        out_specs=pl.BlockSpec((te, d_out), lambda b: (b, 0)),
        out_shape=jax.ShapeDtypeStruct((n, d_out), jnp.float32),
    )(out_in, out_outt, deg_r)
    return out


# speedup vs baseline: 1.4628x; 1.0366x over previous
import functools

import jax
import jax.numpy as jnp
from jax.experimental import pallas as pl
from jax.experimental.pallas import tpu as pltpu


def _pick_tile(n, candidates):
    for c in candidates:
        if n % c == 0:
            return c
    return n


def _yw_body(x_ref, rm_ref, w1_ref, w2_ref, y1_ref, y2t_ref):
    rm = rm_ref[...]
    x = jnp.where(rm > 0.5, x_ref[...], 0.0).astype(jnp.bfloat16)
    w1 = w1_ref[...].astype(jnp.bfloat16)
    w2 = w2_ref[...].astype(jnp.bfloat16)
    y1_ref[...] = jax.lax.dot_general(
        x, w1, (((1,), (0,)), ((), ())),
        preferred_element_type=jnp.float32).astype(jnp.bfloat16)
    y2t_ref[...] = jax.lax.dot_general(
        w2, x, (((0,), (1,)), ((), ())),
        preferred_element_type=jnp.float32).astype(jnp.bfloat16)


def _main_body(ti, tj, a_ref, cm_ref, y1_ref, y2t_ref,
               out_in_ref, out_outt_ref, deg_r_ref):
    j = pl.program_id(0)
    i = pl.program_id(1)
    cm = cm_ref[...].reshape(1, tj)
    a = a_ref[...]
    ab = a.astype(jnp.bfloat16)
    y1 = y1_ref[...]
    isl = pl.ds(i * ti, ti)
    y2t = y2t_ref[...]

    c_in = jax.lax.dot_general(
        ab, y1, (((1,), (0,)), ((), ())), preferred_element_type=jnp.float32)
    c_outt = jax.lax.dot_general(
        y2t, ab, (((1,), (0,)), ((), ())), preferred_element_type=jnp.float32)
    rs = jnp.sum(a * cm, axis=1, keepdims=True)

    @pl.when(j == 0)
    def _():
        out_in_ref[isl, :] = c_in
        deg_r_ref[isl, :] = rs

    @pl.when(j > 0)
    def _():
        out_in_ref[isl, :] += c_in
        deg_r_ref[isl, :] += rs

    @pl.when(i == 0)
    def _():
        out_outt_ref[...] = c_outt

    @pl.when(i > 0)
    def _():
        out_outt_ref[...] += c_outt


def _epi_body(d_out, oi_ref, oot_ref, dr_ref, out_ref):
    r1 = 1.0 / jnp.clip(dr_ref[...], 1e-6, None)
    oota = oot_ref[...]
    r2 = 1.0 / jnp.clip(oota[d_out:, :], 1e-6, None)
    oot = oota[:d_out, :] * r2
    out_ref[...] = oi_ref[...] * r1 + oot.T


def kernel(X, A, W_in, W_out):
    n, d_in = X.shape
    d_out = W_in.shape[1]

    if n >= 2048:
        ti, tj = 1024, 2048
    else:
        ti = tj = n
    nj = -(-n // tj)
    ni = -(-n // ti)
    n_pad = nj * tj
    assert ni * ti == n_pad
    valid = jnp.pad(jnp.ones((n,), jnp.float32), (0, n_pad - n))
    col_valid = valid.reshape(nj, 1, tj)
    row_valid = valid.reshape(n_pad, 1)

    y1, y2t = pl.pallas_call(
        _yw_body,
        grid=(ni,),
        in_specs=[
            pl.BlockSpec((ti, d_in), lambda b: (b, 0)),
            pl.BlockSpec((ti, 1), lambda b: (b, 0)),
            pl.BlockSpec((d_in, d_out), lambda b: (0, 0)),
            pl.BlockSpec((d_in, d_out), lambda b: (0, 0)),
        ],
        out_specs=[
            pl.BlockSpec((ti, d_out), lambda b: (b, 0)),
            pl.BlockSpec((d_out, ti), lambda b: (0, b)),
        ],
        out_shape=[
            jax.ShapeDtypeStruct((n_pad, d_out), jnp.bfloat16),
            jax.ShapeDtypeStruct((d_out, n_pad), jnp.bfloat16),
        ],
    )(X, row_valid, W_in, W_out)

    y2t_aug = jnp.concatenate(
        [y2t, valid.reshape(1, n_pad).astype(jnp.bfloat16)], axis=0)
    out_in, out_outt, deg_r = pl.pallas_call(
        functools.partial(_main_body, ti, tj),
        grid=(nj, ni),
        in_specs=[
            pl.BlockSpec((ti, tj), lambda j, i: (i, j)),
            pl.BlockSpec((1, 1, tj), lambda j, i: (j, 0, 0)),
            pl.BlockSpec((tj, d_out), lambda j, i: (j, 0)),
            pl.BlockSpec((d_out + 1, ti), lambda j, i: (0, i)),
        ],
        out_specs=[
            pl.BlockSpec((n_pad, d_out), lambda j, i: (0, 0)),
            pl.BlockSpec((d_out + 1, tj), lambda j, i: (0, j)),
            pl.BlockSpec((n_pad, 1), lambda j, i: (0, 0)),
        ],
        out_shape=[
            jax.ShapeDtypeStruct((n_pad, d_out), jnp.float32),
            jax.ShapeDtypeStruct((d_out + 1, n_pad), jnp.float32),
            jax.ShapeDtypeStruct((n_pad, 1), jnp.float32),
        ],
        compiler_params=pltpu.CompilerParams(
            dimension_semantics=("arbitrary", "arbitrary"),
            vmem_limit_bytes=64 * 1024 * 1024,
        ),
    )(A, col_valid, y1, y2t_aug)

    te = 1920 if n >= 2048 else n
    out = pl.pallas_call(
        functools.partial(_epi_body, d_out),
        grid=(-(-n // te),),
        in_specs=[
            pl.BlockSpec((te, d_out), lambda b: (b, 0)),
            pl.BlockSpec((d_out + 1, te), lambda b: (0, b)),
            pl.BlockSpec((te, 1), lambda b: (b, 0)),
        ],
        out_specs=pl.BlockSpec((te, d_out), lambda b: (b, 0)),
        out_shape=jax.ShapeDtypeStruct((n, d_out), jnp.float32),
    )(out_in, out_outt, deg_r)
    return out
